# baseline (speedup 1.0000x reference)
import functools

import jax
import jax.numpy as jnp
from jax import lax
from jax.experimental import pallas as pl
from jax.experimental.pallas import tpu as pltpu
from jax.experimental.pallas import tpu_sc as plsc

NB = 1000000
D = 32
B = 16384
NC = 2
NS = 16
NW = NC * NS
BPW = B // NW
L = 16
PW = 512
NPAN = 61
SCAN = NPAN * NW * PW
TAIL = NB - SCAN
RW = 128
DUMP = B
NVREG = B // L


def _extract(sv_ref, pv_ref, cbuf, tmp, bidx_ref, g_hbm, p0, cstart, sem,
             iota):
    cend = cstart + PW

    def cond(st):
        return st[1]

    def body(st):
        p, go = st
        sl = pl.ds(p * L, L)
        sv = sv_ref[sl]
        first = sv[0]
        last = sv[L - 1]
        process = (first < cend) & (last >= cstart)

        @pl.when(process)
        def _():
            pv = pv_ref[sl]
            valid = (sv >= cstart) & (sv < cend)
            cols = jnp.clip(sv - cstart, 0, PW - 1)
            b16 = jnp.where(valid, pv, DUMP)
            bidx_ref[...] = b16
            for d in range(D):
                dcol = jnp.full((L,), d, jnp.int32)
                v = plsc.load_gather(cbuf, [dcol, cols])
                plsc.store_scatter(tmp, [iota, dcol], v)
            pltpu.async_copy(tmp, g_hbm.at[bidx_ref], sem).wait()

        adv = last < cend
        p_next = jnp.where(adv, p + 1, p)
        go_next = adv & (p_next < NVREG)
        return p_next, go_next

    p_end, _ = lax.while_loop(cond, body, (p0, jnp.bool_(True)))
    return p_end


def _tail(sv_ref, pv_ref, t_hbm, tmp, bidx_ref, ridx_ref, g_hbm, sem, iota):
    def cond(st):
        return st[1]

    def body(st):
        p, go = st
        sl = pl.ds(p * L, L)
        sv = sv_ref[sl]
        last = sv[L - 1]
        process = last >= SCAN

        @pl.when(process)
        def _():
            pv = pv_ref[sl]
            valid = sv >= SCAN
            rows = jnp.clip(sv - SCAN, 0, TAIL - 1)
            b16 = jnp.where(valid, pv, DUMP)
            ridx_ref[...] = rows
            pltpu.async_copy(t_hbm.at[ridx_ref], tmp, sem).wait()
            bidx_ref[...] = b16
            pltpu.async_copy(tmp, g_hbm.at[bidx_ref], sem).wait()

        p_next = p + 1
        return p_next, p_next < NVREG

    lax.while_loop(cond, body, (jnp.int32(0), jnp.bool_(True)))


def _body1(te_hbm, ce_hbm, si_hbm, pi_hbm, sj_hbm, pj_hbm, tte_hbm, tce_hbm,
           ga_hbm, gb_hbm,
           siv, piv, sjv, pjv, cba, cbb, tmpa, tmpb, bidxa, bidxb, ridx,
           sem, semw):
    wid = lax.axis_index("s") * NC + lax.axis_index("c")

    pltpu.sync_copy(si_hbm, siv)
    pltpu.sync_copy(pi_hbm, piv)
    pltpu.sync_copy(sj_hbm, sjv)
    pltpu.sync_copy(pj_hbm, pjv)

    iota = lax.iota(jnp.int32, L)
    c0 = wid * NPAN

    def panel(k, carry):
        p_i, p_j = carry
        cstart = (c0 + k) * PW
        coff = pl.multiple_of(cstart, PW)
        ha = pltpu.async_copy(te_hbm.at[:, pl.ds(coff, PW)], cba, sem)
        hb = pltpu.async_copy(ce_hbm.at[:, pl.ds(coff, PW)], cbb, sem)
        ha.wait()
        hb.wait()
        p_i = _extract(siv, piv, cba, tmpa, bidxa, ga_hbm, p_i, cstart,
                       semw, iota)
        p_j = _extract(sjv, pjv, cbb, tmpb, bidxb, gb_hbm, p_j, cstart,
                       semw, iota)
        return p_i, p_j

    lax.fori_loop(0, NPAN, panel, (jnp.int32(0), jnp.int32(0)))

    @pl.when(wid == NW - 1)
    def _():
        _tail(siv, piv, tte_hbm, tmpa, bidxa, ridx, ga_hbm, semw, iota)
        _tail(sjv, pjv, tce_hbm, tmpb, bidxb, ridx, gb_hbm, semw, iota)


def _body2(ga_hbm, gb_hbm, ii_hbm, jj_hbm, tb_hbm, cb_hbm, out_hbm,
           ii_v, jj_v, ibv_v, jbv_v, a_v, b_v, tb_v, cb_v, out_v, sem, semb):
    wid = lax.axis_index("s") * NC + lax.axis_index("c")
    base = wid * BPW

    pltpu.sync_copy(ii_hbm.at[pl.ds(base, BPW)], ii_v)
    pltpu.sync_copy(jj_hbm.at[pl.ds(base, BPW)], jj_v)

    def mkrows(g, carry):
        sl = pl.ds(g * L, L)
        ibv_v[sl] = lax.shift_right_logical(ii_v[sl], 7)
        jbv_v[sl] = lax.shift_right_logical(jj_v[sl], 7)
        return carry

    lax.fori_loop(0, BPW // L, mkrows, 0)

    iota = lax.iota(jnp.int32, L)
    CH = 128

    def chunk(c, carry):
        csl = pl.ds(c * CH, CH)
        boff = pl.multiple_of(base + c * CH, CH)
        h1 = pltpu.async_copy(ga_hbm.at[pl.ds(boff, CH), :], a_v, sem)
        h2 = pltpu.async_copy(gb_hbm.at[pl.ds(boff, CH), :], b_v, sem)
        hb1 = pltpu.async_copy(tb_hbm.at[ibv_v.at[csl]], tb_v, semb)
        hb2 = pltpu.async_copy(cb_hbm.at[jbv_v.at[csl]], cb_v, semb)
        h1.wait()
        h2.wait()
        hb1.wait()
        hb2.wait()

        def grp(g, carry2):
            p0 = c * CH + g * L
            rows = g * L + iota
            ii16 = ii_v[pl.ds(p0, L)]
            jj16 = jj_v[pl.ds(p0, L)]
            acc = plsc.load_gather(tb_v, [rows, ii16 & (RW - 1)])
            acc = acc + plsc.load_gather(cb_v, [rows, jj16 & (RW - 1)])
            for d in range(D):
                dcol = jnp.full((L,), d, jnp.int32)
                va = plsc.load_gather(a_v, [rows, dcol])
                vb = plsc.load_gather(b_v, [rows, dcol])
                acc = acc + va * vb
            out_v[pl.ds(p0, L)] = acc
            return carry2

        lax.fori_loop(0, CH // L, grp, 0)
        return carry

    lax.fori_loop(0, BPW // CH, chunk, 0)

    pltpu.sync_copy(out_v, out_hbm.at[pl.ds(base, BPW)])


@jax.jit
def _run(ii, jj, si, pi, sj, pj, te_t, ce_t, tte, tce, tb, cb):
    mesh = plsc.VectorSubcoreMesh(core_axis_name="c", subcore_axis_name="s")
    cp = pltpu.CompilerParams(
        needs_layout_passes=False, use_tc_tiling_on_sc=True)

    k1 = functools.partial(
        pl.kernel,
        mesh=mesh,
        compiler_params=cp,
        out_type=(jax.ShapeDtypeStruct((B + 1, RW), jnp.float32),
                  jax.ShapeDtypeStruct((B + 1, RW), jnp.float32)),
        scratch_types=[
            pltpu.VMEM((B,), jnp.int32),
            pltpu.VMEM((B,), jnp.int32),
            pltpu.VMEM((B,), jnp.int32),
            pltpu.VMEM((B,), jnp.int32),
            pltpu.VMEM((D, PW), jnp.float32),
            pltpu.VMEM((D, PW), jnp.float32),
            pltpu.VMEM((L, RW), jnp.float32),
            pltpu.VMEM((L, RW), jnp.float32),
            pltpu.VMEM((L,), jnp.int32),
            pltpu.VMEM((L,), jnp.int32),
            pltpu.VMEM((L,), jnp.int32),
            pltpu.SemaphoreType.DMA,
            pltpu.SemaphoreType.DMA,
        ],
    )(_body1)
    ga, gb = k1(te_t, ce_t, si, pi, sj, pj, tte, tce)

    k2 = functools.partial(
        pl.kernel,
        mesh=mesh,
        compiler_params=cp,
        out_type=jax.ShapeDtypeStruct((B,), jnp.float32),
        scratch_types=[
            pltpu.VMEM((BPW,), jnp.int32),
            pltpu.VMEM((BPW,), jnp.int32),
            pltpu.VMEM((BPW,), jnp.int32),
            pltpu.VMEM((BPW,), jnp.int32),
            pltpu.VMEM((128, RW), jnp.float32),
            pltpu.VMEM((128, RW), jnp.float32),
            pltpu.VMEM((128, RW), jnp.float32),
            pltpu.VMEM((128, RW), jnp.float32),
            pltpu.VMEM((BPW,), jnp.float32),
            pltpu.SemaphoreType.DMA,
            pltpu.SemaphoreType.DMA,
        ],
    )(_body2)
    return k2(ga, gb, ii, jj, tb, cb)


def kernel(pair, target_emb, context_emb, target_bias, context_bias):
    ii = pair[:, 0].astype(jnp.int32)
    jj = pair[:, 1].astype(jnp.int32)
    pi = jnp.argsort(ii).astype(jnp.int32)
    pj = jnp.argsort(jj).astype(jnp.int32)
    si = ii[pi]
    sj = jj[pj]
    te_t = target_emb.T
    ce_t = context_emb.T
    tte = jnp.pad(target_emb[SCAN:], ((0, 0), (0, RW - D)))
    tce = jnp.pad(context_emb[SCAN:], ((0, 0), (0, RW - D)))
    npad = -(-NB // RW) * RW
    tb = jnp.pad(target_bias.reshape(-1), (0, npad - NB)).reshape(-1, RW)
    cb = jnp.pad(context_bias.reshape(-1), (0, npad - NB)).reshape(-1, RW)
    out = _run(ii, jj, si, pi, sj, pj, te_t, ce_t, tte, tce, tb, cb)
    return out.reshape(B, 1)


# speedup vs baseline: 1.0228x; 1.0228x over previous
import functools

import jax
import jax.numpy as jnp
from jax import lax
from jax.experimental import pallas as pl
from jax.experimental.pallas import tpu as pltpu
from jax.experimental.pallas import tpu_sc as plsc

NB = 1000000
D = 32
B = 16384
NC = 2
NS = 16
NW = NC * NS
BPW = B // NW
L = 16
PW = 512
NPAN = 61
SCAN = NPAN * NW * PW
TAIL = NB - SCAN
RW = 128
DUMP = B
NVREG = B // L


def _extract(sv_ref, pv_ref, cbuf, tmp, bidx_ref, g_hbm, p0, cstart, sem,
             iota):
    cend = cstart + PW

    def cond(st):
        return st[1]

    def body(st):
        p, go = st
        sl = pl.ds(p * L, L)
        sv = sv_ref[sl]
        first = sv[0]
        last = sv[L - 1]
        process = (first < cend) & (last >= cstart)

        @pl.when(process)
        def _():
            pv = pv_ref[sl]
            valid = (sv >= cstart) & (sv < cend)
            cols = jnp.clip(sv - cstart, 0, PW - 1)
            b16 = jnp.where(valid, pv, DUMP)
            bidx_ref[...] = b16
            for d in range(D):
                dcol = jnp.full((L,), d, jnp.int32)
                v = plsc.load_gather(cbuf, [dcol, cols])
                plsc.store_scatter(tmp, [iota, dcol], v)
            pltpu.async_copy(tmp, g_hbm.at[bidx_ref], sem).wait()

        adv = last < cend
        p_next = jnp.where(adv, p + 1, p)
        go_next = adv & (p_next < NVREG)
        return p_next, go_next

    p_end, _ = lax.while_loop(cond, body, (p0, jnp.bool_(True)))
    return p_end


def _tail(sv_ref, pv_ref, t_hbm, tmp, bidx_ref, ridx_ref, g_hbm, sem, iota,
          p0):
    def cond(st):
        return st[1]

    def body(st):
        p, go = st
        sl = pl.ds(p * L, L)
        sv = sv_ref[sl]
        last = sv[L - 1]
        process = last >= SCAN

        @pl.when(process)
        def _():
            pv = pv_ref[sl]
            valid = sv >= SCAN
            rows = jnp.clip(sv - SCAN, 0, TAIL - 1)
            b16 = jnp.where(valid, pv, DUMP)
            ridx_ref[...] = rows
            pltpu.async_copy(t_hbm.at[ridx_ref], tmp, sem).wait()
            bidx_ref[...] = b16
            pltpu.async_copy(tmp, g_hbm.at[bidx_ref], sem).wait()

        p_next = p + 1
        return p_next, p_next < NVREG

    lax.while_loop(cond, body, (p0, jnp.bool_(True)))


def _body1(te_hbm, ce_hbm, si_hbm, pi_hbm, sj_hbm, pj_hbm, tte_hbm, tce_hbm,
           ga_hbm, gb_hbm,
           siv, piv, sjv, pjv, cba, cbb, tmpa, tmpb, bidxa, bidxb, ridx,
           sem, semw):
    wid = lax.axis_index("s") * NC + lax.axis_index("c")

    pltpu.sync_copy(si_hbm, siv)
    pltpu.sync_copy(pi_hbm, piv)
    pltpu.sync_copy(sj_hbm, sjv)
    pltpu.sync_copy(pj_hbm, pjv)

    iota = lax.iota(jnp.int32, L)
    c0 = wid * NPAN

    def bsearch(sv_ref, target):
        def step(_, st):
            lo, hi = st
            mid = (lo + hi) // 2
            v = sv_ref[pl.ds(mid * L, L)][L - 1]
            take = v >= target
            return jnp.where(take, lo, mid + 1), jnp.where(take, mid, hi)

        lo, _ = lax.fori_loop(0, 10, step,
                              (jnp.int32(0), jnp.int32(NVREG - 1)))
        return lo

    p_i0 = bsearch(siv, c0 * PW)
    p_j0 = bsearch(sjv, c0 * PW)

    def panel(k, carry):
        p_i, p_j = carry
        cstart = (c0 + k) * PW
        coff = pl.multiple_of(cstart, PW)
        ha = pltpu.async_copy(te_hbm.at[:, pl.ds(coff, PW)], cba, sem)
        hb = pltpu.async_copy(ce_hbm.at[:, pl.ds(coff, PW)], cbb, sem)
        ha.wait()
        hb.wait()
        p_i = _extract(siv, piv, cba, tmpa, bidxa, ga_hbm, p_i, cstart,
                       semw, iota)
        p_j = _extract(sjv, pjv, cbb, tmpb, bidxb, gb_hbm, p_j, cstart,
                       semw, iota)
        return p_i, p_j

    lax.fori_loop(0, NPAN, panel, (p_i0, p_j0))

    @pl.when(wid == NW - 1)
    def _():
        pt_i = bsearch(siv, SCAN)
        pt_j = bsearch(sjv, SCAN)
        _tail(siv, piv, tte_hbm, tmpa, bidxa, ridx, ga_hbm, semw, iota, pt_i)
        _tail(sjv, pjv, tce_hbm, tmpb, bidxb, ridx, gb_hbm, semw, iota, pt_j)


def _body2(ga_hbm, gb_hbm, ii_hbm, jj_hbm, tb_hbm, cb_hbm, out_hbm,
           ii_v, jj_v, ibv_v, jbv_v, a_v, b_v, tb_v, cb_v, out_v, sem, semb):
    wid = lax.axis_index("s") * NC + lax.axis_index("c")
    base = wid * BPW

    pltpu.sync_copy(ii_hbm.at[pl.ds(base, BPW)], ii_v)
    pltpu.sync_copy(jj_hbm.at[pl.ds(base, BPW)], jj_v)

    def mkrows(g, carry):
        sl = pl.ds(g * L, L)
        ibv_v[sl] = lax.shift_right_logical(ii_v[sl], 7)
        jbv_v[sl] = lax.shift_right_logical(jj_v[sl], 7)
        return carry

    lax.fori_loop(0, BPW // L, mkrows, 0)

    iota = lax.iota(jnp.int32, L)
    CH = 128

    def chunk(c, carry):
        csl = pl.ds(c * CH, CH)
        boff = pl.multiple_of(base + c * CH, CH)
        h1 = pltpu.async_copy(ga_hbm.at[pl.ds(boff, CH), :], a_v, sem)
        h2 = pltpu.async_copy(gb_hbm.at[pl.ds(boff, CH), :], b_v, sem)
        hb1 = pltpu.async_copy(tb_hbm.at[ibv_v.at[csl]], tb_v, semb)
        hb2 = pltpu.async_copy(cb_hbm.at[jbv_v.at[csl]], cb_v, semb)
        h1.wait()
        h2.wait()
        hb1.wait()
        hb2.wait()

        def grp(g, carry2):
            p0 = c * CH + g * L
            rows = g * L + iota
            ii16 = ii_v[pl.ds(p0, L)]
            jj16 = jj_v[pl.ds(p0, L)]
            acc = plsc.load_gather(tb_v, [rows, ii16 & (RW - 1)])
            acc = acc + plsc.load_gather(cb_v, [rows, jj16 & (RW - 1)])
            for d in range(D):
                dcol = jnp.full((L,), d, jnp.int32)
                va = plsc.load_gather(a_v, [rows, dcol])
                vb = plsc.load_gather(b_v, [rows, dcol])
                acc = acc + va * vb
            out_v[pl.ds(p0, L)] = acc
            return carry2

        lax.fori_loop(0, CH // L, grp, 0)
        return carry

    lax.fori_loop(0, BPW // CH, chunk, 0)

    pltpu.sync_copy(out_v, out_hbm.at[pl.ds(base, BPW)])


@jax.jit
def _run(ii, jj, si, pi, sj, pj, te_t, ce_t, tte, tce, tb, cb):
    mesh = plsc.VectorSubcoreMesh(core_axis_name="c", subcore_axis_name="s")
    cp = pltpu.CompilerParams(
        needs_layout_passes=False, use_tc_tiling_on_sc=True)

    k1 = functools.partial(
        pl.kernel,
        mesh=mesh,
        compiler_params=cp,
        out_type=(jax.ShapeDtypeStruct((B + 1, RW), jnp.float32),
                  jax.ShapeDtypeStruct((B + 1, RW), jnp.float32)),
        scratch_types=[
            pltpu.VMEM((B,), jnp.int32),
            pltpu.VMEM((B,), jnp.int32),
            pltpu.VMEM((B,), jnp.int32),
            pltpu.VMEM((B,), jnp.int32),
            pltpu.VMEM((D, PW), jnp.float32),
            pltpu.VMEM((D, PW), jnp.float32),
            pltpu.VMEM((L, RW), jnp.float32),
            pltpu.VMEM((L, RW), jnp.float32),
            pltpu.VMEM((L,), jnp.int32),
            pltpu.VMEM((L,), jnp.int32),
            pltpu.VMEM((L,), jnp.int32),
            pltpu.SemaphoreType.DMA,
            pltpu.SemaphoreType.DMA,
        ],
    )(_body1)
    ga, gb = k1(te_t, ce_t, si, pi, sj, pj, tte, tce)

    k2 = functools.partial(
        pl.kernel,
        mesh=mesh,
        compiler_params=cp,
        out_type=jax.ShapeDtypeStruct((B,), jnp.float32),
        scratch_types=[
            pltpu.VMEM((BPW,), jnp.int32),
            pltpu.VMEM((BPW,), jnp.int32),
            pltpu.VMEM((BPW,), jnp.int32),
            pltpu.VMEM((BPW,), jnp.int32),
            pltpu.VMEM((128, RW), jnp.float32),
            pltpu.VMEM((128, RW), jnp.float32),
            pltpu.VMEM((128, RW), jnp.float32),
            pltpu.VMEM((128, RW), jnp.float32),
            pltpu.VMEM((BPW,), jnp.float32),
            pltpu.SemaphoreType.DMA,
            pltpu.SemaphoreType.DMA,
        ],
    )(_body2)
    return k2(ga, gb, ii, jj, tb, cb)


def kernel(pair, target_emb, context_emb, target_bias, context_bias):
    ii = pair[:, 0].astype(jnp.int32)
    jj = pair[:, 1].astype(jnp.int32)
    pi = jnp.argsort(ii).astype(jnp.int32)
    pj = jnp.argsort(jj).astype(jnp.int32)
    si = ii[pi]
    sj = jj[pj]
    te_t = target_emb.T
    ce_t = context_emb.T
    tte = jnp.pad(target_emb[SCAN:], ((0, 0), (0, RW - D)))
    tce = jnp.pad(context_emb[SCAN:], ((0, 0), (0, RW - D)))
    npad = -(-NB // RW) * RW
    tb = jnp.pad(target_bias.reshape(-1), (0, npad - NB)).reshape(-1, RW)
    cb = jnp.pad(context_bias.reshape(-1), (0, npad - NB)).reshape(-1, RW)
    out = _run(ii, jj, si, pi, sj, pj, te_t, ce_t, tte, tce, tb, cb)
    return out.reshape(B, 1)


# speedup vs baseline: 1.7960x; 1.7560x over previous
import functools

import jax
import jax.numpy as jnp
from jax import lax
from jax.experimental import pallas as pl
from jax.experimental.pallas import tpu as pltpu
from jax.experimental.pallas import tpu_sc as plsc

NB = 1000000
D = 32
B = 16384
NC = 2
NS = 16
NW = NC * NS
BPW = B // NW
CHUNK = 128
NCH = BPW // CHUNK
L = 16


def _body(ii_hbm, jj_hbm, te_hbm, ce_hbm, tb_hbm, cb_hbm, out_hbm,
          ii_v, jj_v, a_v, b_v, tb_v, cb_v, out_v, sem):
    wid = lax.axis_index("s") * NC + lax.axis_index("c")
    base = wid * BPW

    for c in range(NCH):
        pltpu.sync_copy(ii_hbm.at[wid * NCH + c], ii_v.at[c])
        pltpu.sync_copy(jj_hbm.at[wid * NCH + c], jj_v.at[c])

    handles = []
    for c in range(NCH):
        sl = pl.ds(c * CHUNK, CHUNK)
        handles.append(pltpu.async_copy(te_hbm.at[ii_v.at[c]], a_v.at[sl], sem))
        handles.append(pltpu.async_copy(ce_hbm.at[jj_v.at[c]], b_v.at[sl], sem))
        handles.append(pltpu.async_copy(tb_hbm.at[ii_v.at[c]], tb_v.at[sl], sem))
        handles.append(pltpu.async_copy(cb_hbm.at[jj_v.at[c]], cb_v.at[sl], sem))
    for h in handles:
        h.wait()

    iota = lax.iota(jnp.int32, L)

    def g_body(g, carry):
        rows = g * L + iota
        acc = tb_v[pl.ds(g * L, L)] + cb_v[pl.ds(g * L, L)]
        for d in range(D):
            dcol = jnp.full((L,), d, jnp.int32)
            va = plsc.load_gather(a_v, [rows, dcol])
            vb = plsc.load_gather(b_v, [rows, dcol])
            acc = acc + va * vb
        out_v[pl.ds(g * L, L)] = acc
        return carry

    lax.fori_loop(0, BPW // L, g_body, 0)

    pltpu.sync_copy(out_v, out_hbm.at[pl.ds(base, BPW)])


@functools.partial(jax.jit, static_argnames=())
def _run(ii, jj, te, ce, tb, cb):
    mesh = plsc.VectorSubcoreMesh(core_axis_name="c", subcore_axis_name="s")
    k = functools.partial(
        pl.kernel,
        mesh=mesh,
        compiler_params=pltpu.CompilerParams(
            needs_layout_passes=False, use_tc_tiling_on_sc=False),
        out_type=jax.ShapeDtypeStruct((B,), jnp.float32),
        scratch_types=[
            pltpu.VMEM((NCH, CHUNK), jnp.int32),
            pltpu.VMEM((NCH, CHUNK), jnp.int32),
            pltpu.VMEM((BPW, D), jnp.float32),
            pltpu.VMEM((BPW, D), jnp.float32),
            pltpu.VMEM((BPW,), jnp.float32),
            pltpu.VMEM((BPW,), jnp.float32),
            pltpu.VMEM((BPW,), jnp.float32),
            pltpu.SemaphoreType.DMA,
        ],
    )(_body)
    return k(ii, jj, te, ce, tb, cb)


def kernel(pair, target_emb, context_emb, target_bias, context_bias):
    ii = pair[:, 0].astype(jnp.int32).reshape(B // CHUNK, CHUNK)
    jj = pair[:, 1].astype(jnp.int32).reshape(B // CHUNK, CHUNK)
    tb = target_bias.reshape(-1)
    cb = context_bias.reshape(-1)
    out = _run(ii, jj, target_emb, context_emb, tb, cb)
    return out.reshape(B, 1)
